# Initial kernel scaffold; baseline (speedup 1.0000x reference)
import jax
import jax.numpy as jnp
from jax.experimental import pallas as pl


def kernel(A_edge_index, A_values, H, W, bias):
    raise NotImplementedError("write your pallas kernel here")


# speedup vs baseline: 4.7412x; 4.7412x over previous
import dataclasses
import functools

import jax
import jax.numpy as jnp
from jax import lax
from jax.experimental import pallas as pl
from jax.experimental.pallas import tpu as pltpu
from jax.experimental.pallas import tpu_sc as plsc

N_NODES = 10000
D = 128
NC = 2
NS = 16
NW = NC * NS
UNIT = 128
LANES = 16
CHUNK_ROWS = 80
N_CHUNKS = N_NODES // CHUNK_ROWS


def _matmul_body(h_ref, w_ref, hw_ref):
    hw_ref[...] = jnp.dot(h_ref[...], w_ref[...],
                          preferred_element_type=jnp.float32)


def _matmul(H, W):
    M = H.shape[0]
    BM = 1000
    return pl.pallas_call(
        _matmul_body,
        grid=(M // BM,),
        in_specs=[
            pl.BlockSpec((BM, D), lambda i: (i, 0)),
            pl.BlockSpec((D, D), lambda i: (0, 0)),
        ],
        out_specs=pl.BlockSpec((BM, D), lambda i: (i, 0)),
        out_shape=jax.ShapeDtypeStruct((M, D), jnp.float32),
    )(H, W)


def _spmm_partials(HW, row, col, val, units_per_tile):
    mesh = plsc.VectorSubcoreMesh(core_axis_name="c", subcore_axis_name="s")
    cp = pltpu.CompilerParams()
    if "needs_layout_passes" in pltpu.CompilerParams.__dataclass_fields__:
        cp = dataclasses.replace(cp, needs_layout_passes=False)

    @functools.partial(
        pl.kernel,
        mesh=mesh,
        compiler_params=cp,
        out_type=jax.ShapeDtypeStruct((NC, N_NODES, D), jnp.float32),
        scratch_types=[
            pltpu.VMEM((UNIT,), jnp.int32),
            pltpu.VMEM((UNIT,), jnp.int32),
            pltpu.VMEM((UNIT,), jnp.float32),
            pltpu.VMEM((UNIT, D), jnp.float32),
            pltpu.VMEM_SHARED((N_NODES, D), jnp.float32),
            pltpu.SemaphoreType.DMA,
        ],
    )
    def spmm(hw_hbm, row_hbm, col_hbm, val_hbm, out_hbm,
             colv, rowv, valv, msgs, acc, sem):
        c = lax.axis_index("c")
        s = lax.axis_index("s")
        wid = s * NC + c

        zeros16 = jnp.zeros((LANES,), jnp.float32)

        @pl.loop(0, UNIT)
        def _(e):
            for j in range(D // LANES):
                msgs[e, pl.ds(j * LANES, LANES)] = zeros16

        @pl.loop(0, (N_CHUNKS + NS - 1) // NS)
        def _(i):
            cid = s + i * NS

            @pl.when(cid < N_CHUNKS)
            def _():
                pltpu.sync_copy(
                    msgs.at[pl.ds(0, CHUNK_ROWS)],
                    acc.at[pl.ds(cid * CHUNK_ROWS, CHUNK_ROWS)])

        plsc.subcore_barrier()

        @pl.loop(0, units_per_tile)
        def _(u):
            base = (wid * units_per_tile + u) * UNIT
            pltpu.sync_copy(col_hbm.at[pl.ds(base, UNIT)], colv)
            pltpu.sync_copy(row_hbm.at[pl.ds(base, UNIT)], rowv)
            pltpu.sync_copy(val_hbm.at[pl.ds(base, UNIT)], valv)
            pltpu.async_copy(hw_hbm.at[colv], msgs, sem).wait()

            @pl.loop(0, UNIT)
            def _(e):
                vv = plsc.load_gather(
                    valv, [jnp.zeros((LANES,), jnp.int32) + e])
                for j in range(D // LANES):
                    sl = pl.ds(j * LANES, LANES)
                    msgs[e, sl] = msgs[e, sl] * vv

            pltpu.sync_copy(msgs, acc.at[rowv], add=True)

        plsc.subcore_barrier()

        @pl.loop(0, (N_CHUNKS + NS - 1) // NS)
        def _(i):
            cid = s + i * NS

            @pl.when(cid < N_CHUNKS)
            def _():
                pltpu.sync_copy(
                    acc.at[pl.ds(cid * CHUNK_ROWS, CHUNK_ROWS)],
                    out_hbm.at[c, pl.ds(cid * CHUNK_ROWS, CHUNK_ROWS)])

    return spmm(HW, row, col, val)


def _finish_body(p_ref, b_ref, o_ref):
    o_ref[...] = jnp.maximum(p_ref[0] + p_ref[1] + b_ref[...], 0.0)


def _finish(partials, bias):
    BM = 1000
    return pl.pallas_call(
        _finish_body,
        grid=(N_NODES // BM,),
        in_specs=[
            pl.BlockSpec((NC, BM, D), lambda i: (0, i, 0)),
            pl.BlockSpec((1, D), lambda i: (0, 0)),
        ],
        out_specs=pl.BlockSpec((BM, D), lambda i: (i, 0)),
        out_shape=jax.ShapeDtypeStruct((N_NODES, D), jnp.float32),
    )(partials, bias.reshape(1, D))


def kernel(A_edge_index, A_values, H, W, bias):
    row = A_edge_index[0]
    col = A_edge_index[1]
    E = row.shape[0]
    grain = NW * UNIT
    E_pad = ((E + grain - 1) // grain) * grain
    pad = E_pad - E
    if pad:
        spread = (jnp.arange(pad, dtype=jnp.int32) * 13) % N_NODES
        row = jnp.concatenate([row, spread])
        col = jnp.concatenate([col, spread])
        val = jnp.concatenate([A_values, jnp.zeros((pad,), jnp.float32)])
    else:
        val = A_values
    units_per_tile = E_pad // (NW * UNIT)

    HW = _matmul(H, W)
    partials = _spmm_partials(HW, row, col, val, units_per_tile)
    return _finish(partials, bias)


# speedup vs baseline: 11.1117x; 2.3436x over previous
import dataclasses
import functools

import jax
import jax.numpy as jnp
from jax import lax
from jax.experimental import pallas as pl
from jax.experimental.pallas import tpu as pltpu
from jax.experimental.pallas import tpu_sc as plsc

N_NODES = 10000
D = 128
NC = 2
NS = 16
NW = NC * NS
UNIT = 128
LANES = 16
CHUNK_ROWS = 80
N_CHUNKS = N_NODES // CHUNK_ROWS


def _matmul_body(h_ref, w_ref, hw_ref):
    hw_ref[...] = jnp.dot(h_ref[...], w_ref[...],
                          preferred_element_type=jnp.float32)


def _matmul(H, W):
    M = H.shape[0]
    BM = 1000
    return pl.pallas_call(
        _matmul_body,
        grid=(M // BM,),
        in_specs=[
            pl.BlockSpec((BM, D), lambda i: (i, 0)),
            pl.BlockSpec((D, D), lambda i: (0, 0)),
        ],
        out_specs=pl.BlockSpec((BM, D), lambda i: (i, 0)),
        out_shape=jax.ShapeDtypeStruct((M, D), jnp.float32),
    )(H, W)


DEPTH = 3


def _spmm_partials(HW, row, col, val, units_per_tile):
    n = units_per_tile
    mesh = plsc.VectorSubcoreMesh(core_axis_name="c", subcore_axis_name="s")
    cp = pltpu.CompilerParams()
    if "needs_layout_passes" in pltpu.CompilerParams.__dataclass_fields__:
        cp = dataclasses.replace(cp, needs_layout_passes=False)

    scratch = (
        [pltpu.VMEM((UNIT,), jnp.int32)] * DEPTH
        + [pltpu.VMEM((UNIT,), jnp.int32)] * DEPTH
        + [pltpu.VMEM((UNIT,), jnp.float32)] * DEPTH
        + [pltpu.VMEM((UNIT,), jnp.int32)] * DEPTH
        + [pltpu.VMEM((UNIT, D), jnp.float32)] * DEPTH
        + [pltpu.VMEM_SHARED((N_NODES, D), jnp.float32)]
        + [pltpu.SemaphoreType.DMA] * (3 * DEPTH)
    )

    @functools.partial(
        pl.kernel,
        mesh=mesh,
        compiler_params=cp,
        out_type=jax.ShapeDtypeStruct((NC, N_NODES, D), jnp.float32),
        scratch_types=scratch,
    )
    def spmm(hw_hbm, row_hbm, col_hbm, val_hbm, out_hbm, *sc):
        colv = sc[0:DEPTH]
        rowv = sc[DEPTH:2 * DEPTH]
        valv = sc[2 * DEPTH:3 * DEPTH]
        rowS = sc[3 * DEPTH:4 * DEPTH]
        msgs = sc[4 * DEPTH:5 * DEPTH]
        acc = sc[5 * DEPTH]
        sem_i = sc[5 * DEPTH + 1:5 * DEPTH + 1 + DEPTH]
        sem_g = sc[5 * DEPTH + 1 + DEPTH:5 * DEPTH + 1 + 2 * DEPTH]
        sem_s = sc[5 * DEPTH + 1 + 2 * DEPTH:5 * DEPTH + 1 + 3 * DEPTH]

        c = lax.axis_index("c")
        s = lax.axis_index("s")
        wid = s * NC + c

        def issue_idx(uu, b):
            base = (wid * n + uu) * UNIT
            pltpu.async_copy(col_hbm.at[pl.ds(base, UNIT)], colv[b], sem_i[b])
            pltpu.async_copy(row_hbm.at[pl.ds(base, UNIT)], rowv[b], sem_i[b])
            pltpu.async_copy(val_hbm.at[pl.ds(base, UNIT)], valv[b], sem_i[b])

        def wait_idx(b):
            src = col_hbm.at[pl.ds(0, UNIT)]
            pltpu.make_async_copy(src, colv[b], sem_i[b]).wait()
            pltpu.make_async_copy(src, rowv[b], sem_i[b]).wait()
            vsrc = val_hbm.at[pl.ds(0, UNIT)]
            pltpu.make_async_copy(vsrc, valv[b], sem_i[b]).wait()

        def issue_gather(b):
            pltpu.async_copy(hw_hbm.at[colv[b]], msgs[b], sem_g[b])

        def wait_gather(b):
            pltpu.make_async_copy(hw_hbm.at[colv[b]], msgs[b],
                                  sem_g[b]).wait()

        def issue_scatter(b):
            for j in range(UNIT // LANES):
                sl = pl.ds(j * LANES, LANES)
                rowS[b][sl] = rowv[b][sl]
            pltpu.async_copy(msgs[b], acc.at[rowS[b]], sem_s[b], add=True)

        def wait_scatter(b):
            pltpu.make_async_copy(msgs[b], acc.at[rowS[b]], sem_s[b]).wait()

        def scale(b):
            @pl.loop(0, UNIT)
            def _(e):
                vv = plsc.load_gather(
                    valv[b], [jnp.zeros((LANES,), jnp.int32) + e])
                for j in range(D // LANES):
                    sl = pl.ds(j * LANES, LANES)
                    msgs[b][e, sl] = msgs[b][e, sl] * vv

        zeros16 = jnp.zeros((LANES,), jnp.float32)

        @pl.loop(0, CHUNK_ROWS)
        def _(e):
            for j in range(D // LANES):
                msgs[0][e, pl.ds(j * LANES, LANES)] = zeros16

        @pl.loop(0, (N_CHUNKS + NS - 1) // NS)
        def _(i):
            cid = s + i * NS

            @pl.when(cid < N_CHUNKS)
            def _():
                pltpu.sync_copy(
                    msgs[0].at[pl.ds(0, CHUNK_ROWS)],
                    acc.at[pl.ds(cid * CHUNK_ROWS, CHUNK_ROWS)])

        plsc.subcore_barrier()

        for b in range(DEPTH):
            issue_idx(b, b)
        wait_idx(0)
        issue_gather(0)
        wait_idx(1)
        issue_gather(1)

        @pl.loop(0, n // DEPTH)
        def _(u):
            for b in range(DEPTH):
                uu = u * DEPTH + b
                wait_gather(b)
                scale(b)
                issue_scatter(b)

                @pl.when(uu + DEPTH < n)
                def _():
                    issue_idx(uu + DEPTH, b)

                @pl.when(uu >= 1)
                def _():
                    wait_scatter((b + DEPTH - 1) % DEPTH)

                @pl.when(uu + 2 < n)
                def _():
                    b2 = (b + 2) % DEPTH
                    wait_idx(b2)
                    issue_gather(b2)

        wait_scatter((n - 1) % DEPTH)
        plsc.subcore_barrier()

        @pl.loop(0, (N_CHUNKS + NS - 1) // NS)
        def _(i):
            cid = s + i * NS

            @pl.when(cid < N_CHUNKS)
            def _():
                pltpu.sync_copy(
                    acc.at[pl.ds(cid * CHUNK_ROWS, CHUNK_ROWS)],
                    out_hbm.at[c, pl.ds(cid * CHUNK_ROWS, CHUNK_ROWS)])

    return spmm(HW, row, col, val)


def _finish_body(p_ref, b_ref, o_ref):
    o_ref[...] = jnp.maximum(p_ref[0] + p_ref[1] + b_ref[...], 0.0)


def _finish(partials, bias):
    BM = 1000
    return pl.pallas_call(
        _finish_body,
        grid=(N_NODES // BM,),
        in_specs=[
            pl.BlockSpec((NC, BM, D), lambda i: (0, i, 0)),
            pl.BlockSpec((1, D), lambda i: (0, 0)),
        ],
        out_specs=pl.BlockSpec((BM, D), lambda i: (i, 0)),
        out_shape=jax.ShapeDtypeStruct((N_NODES, D), jnp.float32),
    )(partials, bias.reshape(1, D))


def kernel(A_edge_index, A_values, H, W, bias):
    row = A_edge_index[0]
    col = A_edge_index[1]
    E = row.shape[0]
    grain = NW * UNIT * DEPTH
    E_pad = ((E + grain - 1) // grain) * grain
    pad = E_pad - E
    if pad:
        spread = (jnp.arange(pad, dtype=jnp.int32) * 13) % N_NODES
        row = jnp.concatenate([row, spread])
        col = jnp.concatenate([col, spread])
        val = jnp.concatenate([A_values, jnp.zeros((pad,), jnp.float32)])
    else:
        val = A_values
    units_per_tile = E_pad // (NW * UNIT)

    HW = _matmul(H, W)
    partials = _spmm_partials(HW, row, col, val, units_per_tile)
    return _finish(partials, bias)


# speedup vs baseline: 11.1674x; 1.0050x over previous
import dataclasses
import functools

import jax
import jax.numpy as jnp
from jax import lax
from jax.experimental import pallas as pl
from jax.experimental.pallas import tpu as pltpu
from jax.experimental.pallas import tpu_sc as plsc

N_NODES = 10000
D = 128
NC = 2
NS = 16
NW = NC * NS
UNIT = 128
LANES = 16
CHUNK_ROWS = 80
N_CHUNKS = N_NODES // CHUNK_ROWS


def _matmul_body(h_ref, w_ref, hw_ref):
    hw_ref[...] = jnp.dot(h_ref[...], w_ref[...],
                          preferred_element_type=jnp.float32)


def _matmul(H, W):
    M = H.shape[0]
    BM = 1000
    return pl.pallas_call(
        _matmul_body,
        grid=(M // BM,),
        in_specs=[
            pl.BlockSpec((BM, D), lambda i: (i, 0)),
            pl.BlockSpec((D, D), lambda i: (0, 0)),
        ],
        out_specs=pl.BlockSpec((BM, D), lambda i: (i, 0)),
        out_shape=jax.ShapeDtypeStruct((M, D), jnp.float32),
    )(H, W)


DEPTH = 3


def _spmm_partials(HW, row, col, val, units_per_tile):
    n = units_per_tile
    mesh = plsc.VectorSubcoreMesh(core_axis_name="c", subcore_axis_name="s")
    cp = pltpu.CompilerParams()
    if "needs_layout_passes" in pltpu.CompilerParams.__dataclass_fields__:
        cp = dataclasses.replace(cp, needs_layout_passes=False)

    scratch = (
        [pltpu.VMEM((UNIT,), jnp.int32)] * DEPTH
        + [pltpu.VMEM((UNIT,), jnp.int32)] * DEPTH
        + [pltpu.VMEM((UNIT,), jnp.float32)] * DEPTH
        + [pltpu.VMEM((UNIT,), jnp.int32)] * DEPTH
        + [pltpu.VMEM((UNIT, D), jnp.float32)] * DEPTH
        + [pltpu.VMEM_SHARED((N_NODES, D), jnp.float32)]
        + [pltpu.SemaphoreType.DMA] * (3 * DEPTH)
    )

    @functools.partial(
        pl.kernel,
        mesh=mesh,
        compiler_params=cp,
        out_type=jax.ShapeDtypeStruct((NC, N_NODES, D), jnp.float32),
        scratch_types=scratch,
    )
    def spmm(hw_hbm, row_hbm, col_hbm, val_hbm, out_hbm, *sc):
        colv = sc[0:DEPTH]
        rowv = sc[DEPTH:2 * DEPTH]
        valv = sc[2 * DEPTH:3 * DEPTH]
        rowS = sc[3 * DEPTH:4 * DEPTH]
        msgs = sc[4 * DEPTH:5 * DEPTH]
        acc = sc[5 * DEPTH]
        sem_i = sc[5 * DEPTH + 1:5 * DEPTH + 1 + DEPTH]
        sem_g = sc[5 * DEPTH + 1 + DEPTH:5 * DEPTH + 1 + 2 * DEPTH]
        sem_s = sc[5 * DEPTH + 1 + 2 * DEPTH:5 * DEPTH + 1 + 3 * DEPTH]

        c = lax.axis_index("c")
        s = lax.axis_index("s")
        wid = s * NC + c

        def issue_idx(uu, b):
            base = (wid * n + uu) * UNIT
            pltpu.async_copy(col_hbm.at[pl.ds(base, UNIT)], colv[b], sem_i[b])
            pltpu.async_copy(row_hbm.at[pl.ds(base, UNIT)], rowv[b], sem_i[b])
            pltpu.async_copy(val_hbm.at[pl.ds(base, UNIT)], valv[b], sem_i[b])

        def wait_idx(b):
            src = col_hbm.at[pl.ds(0, UNIT)]
            pltpu.make_async_copy(src, colv[b], sem_i[b]).wait()
            pltpu.make_async_copy(src, rowv[b], sem_i[b]).wait()
            vsrc = val_hbm.at[pl.ds(0, UNIT)]
            pltpu.make_async_copy(vsrc, valv[b], sem_i[b]).wait()

        def issue_gather(b):
            pltpu.async_copy(hw_hbm.at[colv[b]], msgs[b], sem_g[b])

        def wait_gather(b):
            pltpu.make_async_copy(hw_hbm.at[colv[b]], msgs[b],
                                  sem_g[b]).wait()

        def issue_scatter(b):
            for j in range(UNIT // LANES):
                sl = pl.ds(j * LANES, LANES)
                rowS[b][sl] = rowv[b][sl]
            pltpu.async_copy(msgs[b], acc.at[rowS[b]], sem_s[b], add=True)

        def wait_scatter(b):
            pltpu.make_async_copy(msgs[b], acc.at[rowS[b]], sem_s[b]).wait()

        def scale(b):
            @pl.loop(0, UNIT, unroll=4)
            def _(e):
                vv = plsc.load_gather(
                    valv[b], [jnp.zeros((LANES,), jnp.int32) + e])
                for j in range(D // LANES):
                    sl = pl.ds(j * LANES, LANES)
                    msgs[b][e, sl] = msgs[b][e, sl] * vv

        zeros16 = jnp.zeros((LANES,), jnp.float32)

        @pl.loop(0, CHUNK_ROWS)
        def _(e):
            for j in range(D // LANES):
                msgs[0][e, pl.ds(j * LANES, LANES)] = zeros16

        @pl.loop(0, (N_CHUNKS + NS - 1) // NS)
        def _(i):
            cid = s + i * NS

            @pl.when(cid < N_CHUNKS)
            def _():
                pltpu.sync_copy(
                    msgs[0].at[pl.ds(0, CHUNK_ROWS)],
                    acc.at[pl.ds(cid * CHUNK_ROWS, CHUNK_ROWS)])

        plsc.subcore_barrier()

        for b in range(DEPTH):
            issue_idx(b, b)
        wait_idx(0)
        issue_gather(0)
        wait_idx(1)
        issue_gather(1)

        @pl.loop(0, n // DEPTH)
        def _(u):
            for b in range(DEPTH):
                uu = u * DEPTH + b
                wait_gather(b)
                scale(b)
                issue_scatter(b)

                @pl.when(uu + DEPTH < n)
                def _():
                    issue_idx(uu + DEPTH, b)

                @pl.when(uu >= 1)
                def _():
                    wait_scatter((b + DEPTH - 1) % DEPTH)

                @pl.when(uu + 2 < n)
                def _():
                    b2 = (b + 2) % DEPTH
                    wait_idx(b2)
                    issue_gather(b2)

        wait_scatter((n - 1) % DEPTH)
        plsc.subcore_barrier()

        @pl.loop(0, (N_CHUNKS + NS - 1) // NS)
        def _(i):
            cid = s + i * NS

            @pl.when(cid < N_CHUNKS)
            def _():
                pltpu.sync_copy(
                    acc.at[pl.ds(cid * CHUNK_ROWS, CHUNK_ROWS)],
                    out_hbm.at[c, pl.ds(cid * CHUNK_ROWS, CHUNK_ROWS)])

    return spmm(HW, row, col, val)


def _finish_body(p_ref, b_ref, o_ref):
    o_ref[...] = jnp.maximum(p_ref[0] + p_ref[1] + b_ref[...], 0.0)


def _finish(partials, bias):
    BM = 1000
    return pl.pallas_call(
        _finish_body,
        grid=(N_NODES // BM,),
        in_specs=[
            pl.BlockSpec((NC, BM, D), lambda i: (0, i, 0)),
            pl.BlockSpec((1, D), lambda i: (0, 0)),
        ],
        out_specs=pl.BlockSpec((BM, D), lambda i: (i, 0)),
        out_shape=jax.ShapeDtypeStruct((N_NODES, D), jnp.float32),
    )(partials, bias.reshape(1, D))


def kernel(A_edge_index, A_values, H, W, bias):
    row = A_edge_index[0]
    col = A_edge_index[1]
    E = row.shape[0]
    grain = NW * UNIT * DEPTH
    E_pad = ((E + grain - 1) // grain) * grain
    pad = E_pad - E
    if pad:
        spread = (jnp.arange(pad, dtype=jnp.int32) * 13) % N_NODES
        row = jnp.concatenate([row, spread])
        col = jnp.concatenate([col, spread])
        val = jnp.concatenate([A_values, jnp.zeros((pad,), jnp.float32)])
    else:
        val = A_values
    units_per_tile = E_pad // (NW * UNIT)

    HW = _matmul(H, W)
    partials = _spmm_partials(HW, row, col, val, units_per_tile)
    return _finish(partials, bias)


# speedup vs baseline: 11.6266x; 1.0411x over previous
import dataclasses
import functools

import jax
import jax.numpy as jnp
from jax import lax
from jax.experimental import pallas as pl
from jax.experimental.pallas import tpu as pltpu
from jax.experimental.pallas import tpu_sc as plsc

N_NODES = 10000
D = 128
NC = 2
NS = 16
NW = NC * NS
UNIT = 128
LANES = 16
CHUNK_ROWS = 80
N_CHUNKS = N_NODES // CHUNK_ROWS


DEPTH = 3


def _spmm_partials(HW, row, col, val, units_per_tile):
    n = units_per_tile
    mesh = plsc.VectorSubcoreMesh(core_axis_name="c", subcore_axis_name="s")
    cp = pltpu.CompilerParams()
    if "needs_layout_passes" in pltpu.CompilerParams.__dataclass_fields__:
        cp = dataclasses.replace(cp, needs_layout_passes=False)

    scratch = (
        [pltpu.VMEM((UNIT,), jnp.int32)] * DEPTH
        + [pltpu.VMEM((UNIT,), jnp.int32)] * DEPTH
        + [pltpu.VMEM((UNIT,), jnp.float32)] * DEPTH
        + [pltpu.VMEM((UNIT,), jnp.int32)] * DEPTH
        + [pltpu.VMEM((UNIT, D), jnp.float32)] * DEPTH
        + [pltpu.VMEM_SHARED((N_NODES, D), jnp.float32)]
        + [pltpu.SemaphoreType.DMA] * (3 * DEPTH)
    )

    @functools.partial(
        pl.kernel,
        mesh=mesh,
        compiler_params=cp,
        out_type=jax.ShapeDtypeStruct((NC, N_NODES, D), jnp.float32),
        scratch_types=scratch,
    )
    def spmm(hw_hbm, row_hbm, col_hbm, val_hbm, out_hbm, *sc):
        colv = sc[0:DEPTH]
        rowv = sc[DEPTH:2 * DEPTH]
        valv = sc[2 * DEPTH:3 * DEPTH]
        rowS = sc[3 * DEPTH:4 * DEPTH]
        msgs = sc[4 * DEPTH:5 * DEPTH]
        acc = sc[5 * DEPTH]
        sem_i = sc[5 * DEPTH + 1:5 * DEPTH + 1 + DEPTH]
        sem_g = sc[5 * DEPTH + 1 + DEPTH:5 * DEPTH + 1 + 2 * DEPTH]
        sem_s = sc[5 * DEPTH + 1 + 2 * DEPTH:5 * DEPTH + 1 + 3 * DEPTH]

        c = lax.axis_index("c")
        s = lax.axis_index("s")
        wid = s * NC + c

        def issue_idx(uu, b):
            base = (wid * n + uu) * UNIT
            pltpu.async_copy(col_hbm.at[pl.ds(base, UNIT)], colv[b], sem_i[b])
            pltpu.async_copy(row_hbm.at[pl.ds(base, UNIT)], rowv[b], sem_i[b])
            pltpu.async_copy(val_hbm.at[pl.ds(base, UNIT)], valv[b], sem_i[b])

        def wait_idx(b):
            src = col_hbm.at[pl.ds(0, UNIT)]
            pltpu.make_async_copy(src, colv[b], sem_i[b]).wait()
            pltpu.make_async_copy(src, rowv[b], sem_i[b]).wait()
            vsrc = val_hbm.at[pl.ds(0, UNIT)]
            pltpu.make_async_copy(vsrc, valv[b], sem_i[b]).wait()

        def issue_gather(b):
            pltpu.async_copy(hw_hbm.at[colv[b]], msgs[b], sem_g[b])

        def wait_gather(b):
            pltpu.make_async_copy(hw_hbm.at[colv[b]], msgs[b],
                                  sem_g[b]).wait()

        def issue_scatter(b):
            for j in range(UNIT // LANES):
                sl = pl.ds(j * LANES, LANES)
                rowS[b][sl] = rowv[b][sl]
            pltpu.async_copy(msgs[b], acc.at[rowS[b]], sem_s[b], add=True)

        def wait_scatter(b):
            pltpu.make_async_copy(msgs[b], acc.at[rowS[b]], sem_s[b]).wait()

        def scale(b):
            @pl.loop(0, UNIT)
            def _(e):
                vv = plsc.load_gather(
                    valv[b], [jnp.zeros((LANES,), jnp.int32) + e])
                for j in range(D // LANES):
                    sl = pl.ds(j * LANES, LANES)
                    msgs[b][e, sl] = msgs[b][e, sl] * vv

        zeros16 = jnp.zeros((LANES,), jnp.float32)

        @pl.loop(0, CHUNK_ROWS)
        def _(e):
            for j in range(D // LANES):
                msgs[0][e, pl.ds(j * LANES, LANES)] = zeros16

        @pl.loop(0, (N_CHUNKS + NS - 1) // NS)
        def _(i):
            cid = s + i * NS

            @pl.when(cid < N_CHUNKS)
            def _():
                pltpu.sync_copy(
                    msgs[0].at[pl.ds(0, CHUNK_ROWS)],
                    acc.at[pl.ds(cid * CHUNK_ROWS, CHUNK_ROWS)])

        plsc.subcore_barrier()

        for b in range(DEPTH):
            issue_idx(b, b)
        wait_idx(0)
        issue_gather(0)
        wait_idx(1)
        issue_gather(1)

        @pl.loop(0, n // DEPTH)
        def _(u):
            for b in range(DEPTH):
                uu = u * DEPTH + b
                wait_gather(b)
                scale(b)
                issue_scatter(b)

                @pl.when(uu + DEPTH < n)
                def _():
                    issue_idx(uu + DEPTH, b)

                @pl.when(uu >= 1)
                def _():
                    wait_scatter((b + DEPTH - 1) % DEPTH)

                @pl.when(uu + 2 < n)
                def _():
                    b2 = (b + 2) % DEPTH
                    wait_idx(b2)
                    issue_gather(b2)

        wait_scatter((n - 1) % DEPTH)
        plsc.subcore_barrier()

        @pl.loop(0, (N_CHUNKS + NS - 1) // NS)
        def _(i):
            cid = s + i * NS

            @pl.when(cid < N_CHUNKS)
            def _():
                pltpu.sync_copy(
                    acc.at[pl.ds(cid * CHUNK_ROWS, CHUNK_ROWS)],
                    out_hbm.at[c, pl.ds(cid * CHUNK_ROWS, CHUNK_ROWS)])

    return spmm(HW, row, col, val)


def _finish_body(p_ref, w_ref, b_ref, o_ref):
    x = p_ref[0] + p_ref[1]
    y = jnp.dot(x, w_ref[...], preferred_element_type=jnp.float32)
    o_ref[...] = jnp.maximum(y + b_ref[...], 0.0)


def _finish(partials, W, bias):
    BM = 1000
    return pl.pallas_call(
        _finish_body,
        grid=(N_NODES // BM,),
        in_specs=[
            pl.BlockSpec((NC, BM, D), lambda i: (0, i, 0)),
            pl.BlockSpec((D, D), lambda i: (0, 0)),
            pl.BlockSpec((1, D), lambda i: (0, 0)),
        ],
        out_specs=pl.BlockSpec((BM, D), lambda i: (i, 0)),
        out_shape=jax.ShapeDtypeStruct((N_NODES, D), jnp.float32),
    )(partials, W, bias.reshape(1, D))


def kernel(A_edge_index, A_values, H, W, bias):
    row = A_edge_index[0]
    col = A_edge_index[1]
    E = row.shape[0]
    grain = NW * UNIT * DEPTH
    E_pad = ((E + grain - 1) // grain) * grain
    pad = E_pad - E
    if pad:
        spread = (jnp.arange(pad, dtype=jnp.int32) * 13) % N_NODES
        row = jnp.concatenate([row, spread])
        col = jnp.concatenate([col, spread])
        val = jnp.concatenate([A_values, jnp.zeros((pad,), jnp.float32)])
    else:
        val = A_values
    units_per_tile = E_pad // (NW * UNIT)

    partials = _spmm_partials(H, row, col, val, units_per_tile)
    return _finish(partials, W, bias)


# speedup vs baseline: 11.7654x; 1.0119x over previous
import dataclasses
import functools

import jax
import jax.numpy as jnp
from jax import lax
from jax.experimental import pallas as pl
from jax.experimental.pallas import tpu as pltpu
from jax.experimental.pallas import tpu_sc as plsc

N_NODES = 10000
D = 128
NC = 2
NS = 16
NW = NC * NS
UNIT = 128
LANES = 16
CHUNK_ROWS = 80
N_CHUNKS = N_NODES // CHUNK_ROWS


DEPTH = 3


def _spmm_partials(HW, row, col, val, units_per_tile):
    n = units_per_tile
    mesh = plsc.VectorSubcoreMesh(core_axis_name="c", subcore_axis_name="s")
    cp = pltpu.CompilerParams()
    if "needs_layout_passes" in pltpu.CompilerParams.__dataclass_fields__:
        cp = dataclasses.replace(cp, needs_layout_passes=False)

    scratch = (
        [pltpu.VMEM((UNIT,), jnp.int32)] * DEPTH
        + [pltpu.VMEM((UNIT,), jnp.int32)] * DEPTH
        + [pltpu.VMEM((UNIT,), jnp.float32)] * DEPTH
        + [pltpu.VMEM((UNIT,), jnp.int32)] * DEPTH
        + [pltpu.VMEM((UNIT, D), jnp.float32)] * DEPTH
        + [pltpu.VMEM_SHARED((N_NODES, D), jnp.float32)]
        + [pltpu.SemaphoreType.DMA] * (3 * DEPTH + 1)
    )

    @functools.partial(
        pl.kernel,
        mesh=mesh,
        compiler_params=cp,
        out_type=jax.ShapeDtypeStruct((NC, N_NODES, D), jnp.float32),
        scratch_types=scratch,
    )
    def spmm(hw_hbm, row_hbm, col_hbm, val_hbm, out_hbm, *sc):
        colv = sc[0:DEPTH]
        rowv = sc[DEPTH:2 * DEPTH]
        valv = sc[2 * DEPTH:3 * DEPTH]
        rowS = sc[3 * DEPTH:4 * DEPTH]
        msgs = sc[4 * DEPTH:5 * DEPTH]
        acc = sc[5 * DEPTH]
        sem_i = sc[5 * DEPTH + 1:5 * DEPTH + 1 + DEPTH]
        sem_g = sc[5 * DEPTH + 1 + DEPTH:5 * DEPTH + 1 + 2 * DEPTH]
        sem_s = sc[5 * DEPTH + 1 + 2 * DEPTH:5 * DEPTH + 1 + 3 * DEPTH]
        sem_z = sc[5 * DEPTH + 1 + 3 * DEPTH]
        zbuf = msgs[DEPTH - 1].at[pl.ds(0, CHUNK_ROWS)]

        c = lax.axis_index("c")
        s = lax.axis_index("s")
        wid = s * NC + c

        def issue_idx(uu, b):
            base = (wid * n + uu) * UNIT
            pltpu.async_copy(col_hbm.at[pl.ds(base, UNIT)], colv[b], sem_i[b])
            pltpu.async_copy(row_hbm.at[pl.ds(base, UNIT)], rowv[b], sem_i[b])
            pltpu.async_copy(val_hbm.at[pl.ds(base, UNIT)], valv[b], sem_i[b])

        def wait_idx(b):
            src = col_hbm.at[pl.ds(0, UNIT)]
            pltpu.make_async_copy(src, colv[b], sem_i[b]).wait()
            pltpu.make_async_copy(src, rowv[b], sem_i[b]).wait()
            vsrc = val_hbm.at[pl.ds(0, UNIT)]
            pltpu.make_async_copy(vsrc, valv[b], sem_i[b]).wait()

        def issue_gather(b):
            pltpu.async_copy(hw_hbm.at[colv[b]], msgs[b], sem_g[b])

        def wait_gather(b):
            pltpu.make_async_copy(hw_hbm.at[colv[b]], msgs[b],
                                  sem_g[b]).wait()

        def issue_scatter(b):
            for j in range(UNIT // LANES):
                sl = pl.ds(j * LANES, LANES)
                rowS[b][sl] = rowv[b][sl]
            pltpu.async_copy(msgs[b], acc.at[rowS[b]], sem_s[b], add=True)

        def wait_scatter(b):
            pltpu.make_async_copy(msgs[b], acc.at[rowS[b]], sem_s[b]).wait()

        def scale(b):
            @pl.loop(0, UNIT)
            def _(e):
                vv = plsc.load_gather(
                    valv[b], [jnp.zeros((LANES,), jnp.int32) + e])
                for j in range(D // LANES):
                    sl = pl.ds(j * LANES, LANES)
                    msgs[b][e, sl] = msgs[b][e, sl] * vv

        for b in range(DEPTH):
            issue_idx(b, b)

        zeros16 = jnp.zeros((LANES,), jnp.float32)

        @pl.loop(0, CHUNK_ROWS)
        def _(e):
            for j in range(D // LANES):
                msgs[DEPTH - 1][e, pl.ds(j * LANES, LANES)] = zeros16

        @pl.loop(0, (N_CHUNKS + NS - 1) // NS)
        def _(i):
            cid = s + i * NS

            @pl.when(cid < N_CHUNKS)
            def _():
                pltpu.async_copy(
                    zbuf, acc.at[pl.ds(cid * CHUNK_ROWS, CHUNK_ROWS)], sem_z)

        for b in range(DEPTH - 1):
            wait_idx(b)
            issue_gather(b)

        @pl.loop(0, (N_CHUNKS + NS - 1) // NS)
        def _(i):
            cid = s + i * NS

            @pl.when(cid < N_CHUNKS)
            def _():
                pltpu.make_async_copy(
                    zbuf, acc.at[pl.ds(cid * CHUNK_ROWS, CHUNK_ROWS)],
                    sem_z).wait()

        plsc.subcore_barrier()

        @pl.loop(0, n // DEPTH)
        def _(u):
            for b in range(DEPTH):
                uu = u * DEPTH + b
                bN = (b + DEPTH - 1) % DEPTH
                wait_gather(b)
                scale(b)
                issue_scatter(b)

                @pl.when(uu + DEPTH < n)
                def _():
                    issue_idx(uu + DEPTH, b)

                @pl.when(uu >= 1)
                def _():
                    wait_scatter(bN)

                @pl.when(uu + DEPTH - 1 < n)
                def _():
                    wait_idx(bN)
                    issue_gather(bN)

        wait_scatter((n - 1) % DEPTH)
        plsc.subcore_barrier()

        @pl.loop(0, (N_CHUNKS + NS - 1) // NS)
        def _(i):
            cid = s + i * NS

            @pl.when(cid < N_CHUNKS)
            def _():
                pltpu.async_copy(
                    acc.at[pl.ds(cid * CHUNK_ROWS, CHUNK_ROWS)],
                    out_hbm.at[c, pl.ds(cid * CHUNK_ROWS, CHUNK_ROWS)], sem_z)

        @pl.loop(0, (N_CHUNKS + NS - 1) // NS)
        def _(i):
            cid = s + i * NS

            @pl.when(cid < N_CHUNKS)
            def _():
                pltpu.make_async_copy(
                    acc.at[pl.ds(cid * CHUNK_ROWS, CHUNK_ROWS)],
                    out_hbm.at[c, pl.ds(cid * CHUNK_ROWS, CHUNK_ROWS)],
                    sem_z).wait()

    return spmm(HW, row, col, val)


def _finish_body(p_ref, w_ref, b_ref, o_ref):
    x = p_ref[0] + p_ref[1]
    y = jnp.dot(x, w_ref[...], preferred_element_type=jnp.float32)
    o_ref[...] = jnp.maximum(y + b_ref[...], 0.0)


def _finish(partials, W, bias):
    BM = 1000
    return pl.pallas_call(
        _finish_body,
        grid=(N_NODES // BM,),
        in_specs=[
            pl.BlockSpec((NC, BM, D), lambda i: (0, i, 0)),
            pl.BlockSpec((D, D), lambda i: (0, 0)),
            pl.BlockSpec((1, D), lambda i: (0, 0)),
        ],
        out_specs=pl.BlockSpec((BM, D), lambda i: (i, 0)),
        out_shape=jax.ShapeDtypeStruct((N_NODES, D), jnp.float32),
    )(partials, W, bias.reshape(1, D))


def kernel(A_edge_index, A_values, H, W, bias):
    row = A_edge_index[0]
    col = A_edge_index[1]
    E = row.shape[0]
    grain = NW * UNIT * DEPTH
    E_pad = ((E + grain - 1) // grain) * grain
    pad = E_pad - E
    if pad:
        spread = (jnp.arange(pad, dtype=jnp.int32) * 13) % N_NODES
        row = jnp.concatenate([row, spread])
        col = jnp.concatenate([col, spread])
        val = jnp.concatenate([A_values, jnp.zeros((pad,), jnp.float32)])
    else:
        val = A_values
    units_per_tile = E_pad // (NW * UNIT)

    partials = _spmm_partials(H, row, col, val, units_per_tile)
    return _finish(partials, W, bias)
